# Initial kernel scaffold; baseline (speedup 1.0000x reference)
import jax
import jax.numpy as jnp
from jax.experimental import pallas as pl


def kernel(z, W):
    raise NotImplementedError("write your pallas kernel here")


# speedup vs baseline: 1.1241x; 1.1241x over previous
import functools

import jax
import jax.numpy as jnp
from jax import lax
from jax.experimental import pallas as pl
from jax.experimental.pallas import tpu as pltpu
from jax.experimental.pallas import tpu_sc as plsc

N_E = 8192
E_DIM = 256
N_TOK = 8192
BETA = 0.25

BM = 512
BN = 1024
NT = N_TOK // BM


def _argmin_body(z_ref, sz_ref, sw_ref, wt_ref, idx_ref, dsum_ref):
    i = pl.program_id(0)
    z = z_ref[...]
    sz = sz_ref[...]

    dmin = jnp.full((BM, 1), jnp.inf, jnp.float32)
    imin = jnp.zeros((BM, 1), jnp.float32)
    codes = lax.broadcasted_iota(jnp.int32, (BM, BN), 1).astype(jnp.float32)
    for j in range(N_E // BN):
        wt2 = wt_ref[:, j * BN:(j + 1) * BN]
        sw = sw_ref[:, j * BN:(j + 1) * BN]
        mm2 = jnp.dot(z, wt2, preferred_element_type=jnp.float32)
        d = (sz + sw) - mm2
        lmin = jnp.min(d, axis=1, keepdims=True)
        limin = jnp.min(jnp.where(d == lmin, codes, float(BN)), axis=1,
                        keepdims=True) + float(j * BN)
        upd = lmin < dmin
        dmin = jnp.where(upd, lmin, dmin)
        imin = jnp.where(upd, limin, imin)

    idx_ref[...] = imin.astype(jnp.int32)
    s = jnp.sum(dmin).reshape(1, 1)

    @pl.when(i == 0)
    def _():
        dsum_ref[...] = s

    @pl.when(i != 0)
    def _():
        dsum_ref[...] += s


def _tc_argmin(z_flat, sz, sw, wt):
    return pl.pallas_call(
        _argmin_body,
        grid=(NT,),
        in_specs=[
            pl.BlockSpec((BM, E_DIM), lambda i: (i, 0)),
            pl.BlockSpec((BM, 1), lambda i: (i, 0)),
            pl.BlockSpec((1, N_E), lambda i: (0, 0)),
            pl.BlockSpec((E_DIM, N_E), lambda i: (0, 0)),
        ],
        out_specs=[
            pl.BlockSpec((BM, 1), lambda i: (i, 0)),
            pl.BlockSpec((1, 1), lambda i: (0, 0)),
        ],
        out_shape=[
            jax.ShapeDtypeStruct((N_TOK, 1), jnp.int32),
            jax.ShapeDtypeStruct((1, 1), jnp.float32),
        ],
    )(z_flat, sz, sw, wt)


_B_PER_W = N_TOK // 32
_IDX_ROWS = N_TOK // 128


def _sc_body(w_hbm, idx_hbm, zq_hbm, uniq_hbm,
             idx_v, rows_v, sidx_v, ones_v, zeros_v, slice_v, cnt_v,
             table_sh, sem):
    cid = lax.axis_index("c")
    sid = lax.axis_index("s")
    wid = cid * 16 + sid

    pltpu.sync_copy(idx_hbm.at[pl.ds(wid * 2, 2)], idx_v)
    c0 = pltpu.async_copy(w_hbm.at[idx_v.at[0]], rows_v.at[pl.ds(0, 128)],
                          sem)
    c1 = pltpu.async_copy(w_hbm.at[idx_v.at[1]], rows_v.at[pl.ds(128, 128)],
                          sem)
    c0.wait()
    c1.wait()
    pltpu.sync_copy(rows_v, zq_hbm.at[pl.ds(wid * _B_PER_W, _B_PER_W)])

    @pl.when(cid == 0)
    def _():
        for k in range(512 // 16):
            ones_v[pl.ds(k * 16, 16)] = jnp.ones((16,), jnp.int32)
            zeros_v[pl.ds(k * 16, 16)] = jnp.zeros((16,), jnp.int32)
        pltpu.sync_copy(zeros_v, table_sh.at[pl.ds(sid * 512, 512)])
        plsc.subcore_barrier()
        pltpu.sync_copy(idx_hbm.at[pl.ds(sid * 4, 4)], sidx_v)
        for j in range(4):
            pltpu.sync_copy(ones_v.at[pl.ds(j * 128, 128)],
                            table_sh.at[sidx_v.at[j]], add=True)
        plsc.subcore_barrier()
        pltpu.sync_copy(table_sh.at[pl.ds(sid * 512, 512)], slice_v)
        total = jnp.zeros((16,), jnp.int32)
        for k in range(512 // 16):
            x = slice_v[pl.ds(k * 16, 16)]
            total = total + jnp.minimum(x, jnp.ones((16,), jnp.int32))
        cnt_v[...] = total
        pltpu.sync_copy(cnt_v, uniq_hbm.at[sid])


@functools.cache
def _sc_gather_kernel():
    return pl.kernel(
        _sc_body,
        out_type=[
            jax.ShapeDtypeStruct((N_TOK, E_DIM), jnp.float32),
            jax.ShapeDtypeStruct((16, 16), jnp.int32),
        ],
        mesh=plsc.VectorSubcoreMesh(core_axis_name="c",
                                    subcore_axis_name="s"),
        scratch_types=[
            pltpu.VMEM((2, 128), jnp.int32),
            pltpu.VMEM((_B_PER_W, E_DIM), jnp.float32),
            pltpu.VMEM((4, 128), jnp.int32),
            pltpu.VMEM((512,), jnp.int32),
            pltpu.VMEM((512,), jnp.int32),
            pltpu.VMEM((512,), jnp.int32),
            pltpu.VMEM((16,), jnp.int32),
            pltpu.VMEM_SHARED((N_E,), jnp.int32),
            pltpu.SemaphoreType.DMA,
        ],
    )


def _sc_gather(w, idx2d):
    return _sc_gather_kernel()(w, idx2d)


def kernel(z, W):
    zp = jnp.transpose(z, (0, 2, 3, 4, 1))
    z_flat = zp.reshape(-1, E_DIM)
    sz = jnp.sum(z_flat ** 2, axis=1, keepdims=True)
    sw = jnp.sum(W ** 2, axis=1).reshape(1, N_E)
    z_bf = z_flat.astype(jnp.bfloat16)
    wt_bf = (2.0 * W).astype(jnp.bfloat16).T

    idx2d, dsum = _tc_argmin(z_bf, sz, sw, wt_bf)
    idx = idx2d.reshape(N_TOK)

    zq_flat, uniq_rows = _sc_gather(W, idx.reshape(_IDX_ROWS, 128))
    unique = jnp.sum(uniq_rows)

    m = dsum[0, 0] / jnp.float32(N_TOK * E_DIM)
    loss = jnp.float32(BETA) * m + m

    zq_st = z_flat + (zq_flat - z_flat)
    z_q = jnp.transpose(zq_st.reshape(zp.shape), (0, 4, 1, 2, 3))
    return (z_q, loss, unique, idx)


# speedup vs baseline: 1.1767x; 1.0468x over previous
import functools

import jax
import jax.numpy as jnp
from jax import lax
from jax.experimental import pallas as pl
from jax.experimental.pallas import tpu as pltpu
from jax.experimental.pallas import tpu_sc as plsc

N_E = 8192
E_DIM = 256
N_TOK = 8192
BETA = 0.25

BM = 256
BN = 512
NT = N_TOK // BM


def _argmin_body(z_ref, sz_ref, sw_ref, wt_ref, idx_ref, dsum_ref):
    i = pl.program_id(0)
    z = z_ref[...]
    sz = sz_ref[...]

    lane = lax.broadcasted_iota(jnp.int32, (BM, 128), 1).astype(jnp.float32)
    vmin = jnp.full((BM, 128), jnp.inf, jnp.float32)
    iml = jnp.zeros((BM, 128), jnp.float32)
    for j in range(N_E // BN):
        wt2 = wt_ref[:, j * BN:(j + 1) * BN]
        mm2 = jnp.dot(z, wt2, preferred_element_type=jnp.float32)
        for c in range(BN // 128):
            col = j * BN + c * 128
            dcol = (sz + sw_ref[:, col:col + 128]) - mm2[:, c * 128:(c + 1) * 128]
            upd = dcol < vmin
            vmin = jnp.where(upd, dcol, vmin)
            iml = jnp.where(upd, lane + float(col), iml)
    dmin = jnp.min(vmin, axis=1, keepdims=True)
    imin = jnp.min(jnp.where(vmin == dmin, iml, float(N_E)), axis=1,
                   keepdims=True)
    idx_ref[...] = imin.astype(jnp.int32)
    s = jnp.sum(dmin).reshape(1, 1)

    @pl.when(i == 0)
    def _():
        dsum_ref[...] = s

    @pl.when(i != 0)
    def _():
        dsum_ref[...] += s


def _tc_argmin(z_flat, sz, sw, wt):
    return pl.pallas_call(
        _argmin_body,
        grid=(NT,),
        in_specs=[
            pl.BlockSpec((BM, E_DIM), lambda i: (i, 0)),
            pl.BlockSpec((BM, 1), lambda i: (i, 0)),
            pl.BlockSpec((1, N_E), lambda i: (0, 0)),
            pl.BlockSpec((E_DIM, N_E), lambda i: (0, 0)),
        ],
        out_specs=[
            pl.BlockSpec((BM, 1), lambda i: (i, 0)),
            pl.BlockSpec((1, 1), lambda i: (0, 0)),
        ],
        out_shape=[
            jax.ShapeDtypeStruct((N_TOK, 1), jnp.int32),
            jax.ShapeDtypeStruct((1, 1), jnp.float32),
        ],
    )(z_flat, sz, sw, wt)


_B_PER_W = N_TOK // 32
_IDX_ROWS = N_TOK // 128


def _sc_body(w_hbm, idx_hbm, zq_hbm, uniq_hbm,
             idx_v, rows_v, sidx_v, ones_v, zeros_v, slice_v, cnt_v,
             table_sh, sem):
    cid = lax.axis_index("c")
    sid = lax.axis_index("s")
    wid = cid * 16 + sid

    pltpu.sync_copy(idx_hbm.at[pl.ds(wid * 2, 2)], idx_v)
    c0 = pltpu.async_copy(w_hbm.at[idx_v.at[0]], rows_v.at[pl.ds(0, 128)],
                          sem)
    c1 = pltpu.async_copy(w_hbm.at[idx_v.at[1]], rows_v.at[pl.ds(128, 128)],
                          sem)
    c0.wait()
    c1.wait()
    pltpu.sync_copy(rows_v, zq_hbm.at[pl.ds(wid * _B_PER_W, _B_PER_W)])

    @pl.when(cid == 0)
    def _():
        for k in range(512 // 16):
            ones_v[pl.ds(k * 16, 16)] = jnp.ones((16,), jnp.int32)
            zeros_v[pl.ds(k * 16, 16)] = jnp.zeros((16,), jnp.int32)
        pltpu.sync_copy(zeros_v, table_sh.at[pl.ds(sid * 512, 512)])
        plsc.subcore_barrier()
        pltpu.sync_copy(idx_hbm.at[pl.ds(sid * 4, 4)], sidx_v)
        for j in range(4):
            pltpu.sync_copy(ones_v.at[pl.ds(j * 128, 128)],
                            table_sh.at[sidx_v.at[j]], add=True)
        plsc.subcore_barrier()
        pltpu.sync_copy(table_sh.at[pl.ds(sid * 512, 512)], slice_v)
        total = jnp.zeros((16,), jnp.int32)
        for k in range(512 // 16):
            x = slice_v[pl.ds(k * 16, 16)]
            total = total + jnp.minimum(x, jnp.ones((16,), jnp.int32))
        cnt_v[...] = total
        pltpu.sync_copy(cnt_v, uniq_hbm.at[sid])


@functools.cache
def _sc_gather_kernel():
    return pl.kernel(
        _sc_body,
        out_type=[
            jax.ShapeDtypeStruct((N_TOK, E_DIM), jnp.float32),
            jax.ShapeDtypeStruct((16, 16), jnp.int32),
        ],
        mesh=plsc.VectorSubcoreMesh(core_axis_name="c",
                                    subcore_axis_name="s"),
        scratch_types=[
            pltpu.VMEM((2, 128), jnp.int32),
            pltpu.VMEM((_B_PER_W, E_DIM), jnp.float32),
            pltpu.VMEM((4, 128), jnp.int32),
            pltpu.VMEM((512,), jnp.int32),
            pltpu.VMEM((512,), jnp.int32),
            pltpu.VMEM((512,), jnp.int32),
            pltpu.VMEM((16,), jnp.int32),
            pltpu.VMEM_SHARED((N_E,), jnp.int32),
            pltpu.SemaphoreType.DMA,
        ],
    )


def _sc_gather(w, idx2d):
    return _sc_gather_kernel()(w, idx2d)


def kernel(z, W):
    zp = jnp.transpose(z, (0, 2, 3, 4, 1))
    z_flat = zp.reshape(-1, E_DIM)
    sz = jnp.sum(z_flat ** 2, axis=1, keepdims=True)
    sw = jnp.sum(W ** 2, axis=1).reshape(1, N_E)
    z_bf = z_flat.astype(jnp.bfloat16)
    wt_bf = (2.0 * W).astype(jnp.bfloat16).T

    idx2d, dsum = _tc_argmin(z_bf, sz, sw, wt_bf)
    idx = idx2d.reshape(N_TOK)

    zq_flat, uniq_rows = _sc_gather(W, idx.reshape(_IDX_ROWS, 128))
    unique = jnp.sum(uniq_rows)

    m = dsum[0, 0] / jnp.float32(N_TOK * E_DIM)
    loss = jnp.float32(BETA) * m + m

    zq_st = z_flat + (zq_flat - z_flat)
    z_q = jnp.transpose(zq_st.reshape(zp.shape), (0, 4, 1, 2, 3))
    return (z_q, loss, unique, idx)


# speedup vs baseline: 1.2761x; 1.0844x over previous
import functools

import jax
import jax.numpy as jnp
from jax import lax
from jax.experimental import pallas as pl
from jax.experimental.pallas import tpu as pltpu
from jax.experimental.pallas import tpu_sc as plsc

N_E = 8192
E_DIM = 256
N_TOK = 8192
BETA = 0.25

BM = 512
BN = 512
NT = N_TOK // BM


def _argmin_body(z_ref, sz_ref, sw_ref, wt_ref, idx_ref, dsum_ref):
    i = pl.program_id(0)
    z = z_ref[...]
    sz = sz_ref[...]

    lane = lax.broadcasted_iota(jnp.int32, (BM, 128), 1).astype(jnp.float32)
    vmin = jnp.full((BM, 128), jnp.inf, jnp.float32)
    iml = jnp.zeros((BM, 128), jnp.float32)
    for j in range(N_E // BN):
        wt2 = wt_ref[:, j * BN:(j + 1) * BN]
        mm2 = jnp.dot(z, wt2, preferred_element_type=jnp.float32)
        for c in range(BN // 128):
            col = j * BN + c * 128
            dcol = (sz + sw_ref[:, col:col + 128]) - mm2[:, c * 128:(c + 1) * 128]
            upd = dcol < vmin
            vmin = jnp.where(upd, dcol, vmin)
            iml = jnp.where(upd, lane + float(col), iml)
    dmin = jnp.min(vmin, axis=1, keepdims=True)
    imin = jnp.min(jnp.where(vmin == dmin, iml, float(N_E)), axis=1,
                   keepdims=True)
    idx_ref[...] = imin.astype(jnp.int32)
    s = jnp.sum(dmin).reshape(1, 1)

    @pl.when(i == 0)
    def _():
        dsum_ref[...] = s

    @pl.when(i != 0)
    def _():
        dsum_ref[...] += s


def _tc_argmin(z_flat, sz, sw, wt):
    return pl.pallas_call(
        _argmin_body,
        grid=(NT,),
        in_specs=[
            pl.BlockSpec((BM, E_DIM), lambda i: (i, 0)),
            pl.BlockSpec((BM, 1), lambda i: (i, 0)),
            pl.BlockSpec((1, N_E), lambda i: (0, 0)),
            pl.BlockSpec((E_DIM, N_E), lambda i: (0, 0)),
        ],
        out_specs=[
            pl.BlockSpec((BM, 1), lambda i: (i, 0)),
            pl.BlockSpec((1, 1), lambda i: (0, 0)),
        ],
        out_shape=[
            jax.ShapeDtypeStruct((N_TOK, 1), jnp.int32),
            jax.ShapeDtypeStruct((1, 1), jnp.float32),
        ],
    )(z_flat, sz, sw, wt)


_B_PER_W = N_TOK // 32
_IDX_ROWS = N_TOK // 128


def _sc_body(w_hbm, idx_hbm, zq_hbm, uniq_hbm,
             idx_v, rows_v, sidx_v, ones_v, zeros_v, slice_v, cnt_v,
             table_sh, sem):
    cid = lax.axis_index("c")
    sid = lax.axis_index("s")
    wid = cid * 16 + sid

    pltpu.sync_copy(idx_hbm.at[pl.ds(wid * 2, 2)], idx_v)
    c0 = pltpu.async_copy(w_hbm.at[idx_v.at[0]], rows_v.at[pl.ds(0, 128)],
                          sem)
    c1 = pltpu.async_copy(w_hbm.at[idx_v.at[1]], rows_v.at[pl.ds(128, 128)],
                          sem)
    c0.wait()
    c1.wait()
    pltpu.sync_copy(rows_v, zq_hbm.at[pl.ds(wid * _B_PER_W, _B_PER_W)])

    @pl.when(cid == 0)
    def _():
        for k in range(512 // 16):
            ones_v[pl.ds(k * 16, 16)] = jnp.ones((16,), jnp.int32)
            zeros_v[pl.ds(k * 16, 16)] = jnp.zeros((16,), jnp.int32)
        pltpu.sync_copy(zeros_v, table_sh.at[pl.ds(sid * 512, 512)])
        plsc.subcore_barrier()
        pltpu.sync_copy(idx_hbm.at[pl.ds(sid * 4, 4)], sidx_v)
        for j in range(4):
            pltpu.sync_copy(ones_v.at[pl.ds(j * 128, 128)],
                            table_sh.at[sidx_v.at[j]], add=True)
        plsc.subcore_barrier()
        pltpu.sync_copy(table_sh.at[pl.ds(sid * 512, 512)], slice_v)
        total = jnp.zeros((16,), jnp.int32)
        for k in range(512 // 16):
            x = slice_v[pl.ds(k * 16, 16)]
            total = total + jnp.minimum(x, jnp.ones((16,), jnp.int32))
        cnt_v[...] = total
        pltpu.sync_copy(cnt_v, uniq_hbm.at[sid])


@functools.cache
def _sc_gather_kernel():
    return pl.kernel(
        _sc_body,
        out_type=[
            jax.ShapeDtypeStruct((N_TOK, E_DIM), jnp.float32),
            jax.ShapeDtypeStruct((16, 16), jnp.int32),
        ],
        mesh=plsc.VectorSubcoreMesh(core_axis_name="c",
                                    subcore_axis_name="s"),
        scratch_types=[
            pltpu.VMEM((2, 128), jnp.int32),
            pltpu.VMEM((_B_PER_W, E_DIM), jnp.float32),
            pltpu.VMEM((4, 128), jnp.int32),
            pltpu.VMEM((512,), jnp.int32),
            pltpu.VMEM((512,), jnp.int32),
            pltpu.VMEM((512,), jnp.int32),
            pltpu.VMEM((16,), jnp.int32),
            pltpu.VMEM_SHARED((N_E,), jnp.int32),
            pltpu.SemaphoreType.DMA,
        ],
    )


def _sc_gather(w, idx2d):
    return _sc_gather_kernel()(w, idx2d)


def kernel(z, W):
    zp = jnp.transpose(z, (0, 2, 3, 4, 1))
    z_flat = zp.reshape(-1, E_DIM)
    sz = jnp.sum(z_flat ** 2, axis=1, keepdims=True)
    sw = jnp.sum(W ** 2, axis=1).reshape(1, N_E)
    z_bf = z_flat.astype(jnp.bfloat16)
    wt_bf = (2.0 * W).astype(jnp.bfloat16).T

    idx2d, dsum = _tc_argmin(z_bf, sz, sw, wt_bf)
    idx = idx2d.reshape(N_TOK)

    zq_flat, uniq_rows = _sc_gather(W, idx.reshape(_IDX_ROWS, 128))
    unique = jnp.sum(uniq_rows)

    m = dsum[0, 0] / jnp.float32(N_TOK * E_DIM)
    loss = jnp.float32(BETA) * m + m

    zq_st = z_flat + (zq_flat - z_flat)
    z_q = jnp.transpose(zq_st.reshape(zp.shape), (0, 4, 1, 2, 3))
    return (z_q, loss, unique, idx)


# speedup vs baseline: 1.2791x; 1.0024x over previous
import functools

import jax
import jax.numpy as jnp
from jax import lax
from jax.experimental import pallas as pl
from jax.experimental.pallas import tpu as pltpu
from jax.experimental.pallas import tpu_sc as plsc

N_E = 8192
E_DIM = 256
N_TOK = 8192
BETA = 0.25

BM = 512
BN = 256
NT = N_TOK // BM


def _argmin_body(z_ref, sz_ref, sw_ref, wt_ref, idx_ref, dsum_ref):
    i = pl.program_id(0)
    z = z_ref[...]
    sz = sz_ref[...]

    lane = lax.broadcasted_iota(jnp.int32, (BM, 128), 1).astype(jnp.float32)
    vmin = jnp.full((BM, 128), jnp.inf, jnp.float32)
    iml = jnp.zeros((BM, 128), jnp.float32)
    for j in range(N_E // BN):
        wt2 = wt_ref[:, j * BN:(j + 1) * BN]
        mm2 = jnp.dot(z, wt2, preferred_element_type=jnp.float32)
        for c in range(BN // 128):
            col = j * BN + c * 128
            dcol = (sz + sw_ref[:, col:col + 128]) - mm2[:, c * 128:(c + 1) * 128]
            upd = dcol < vmin
            vmin = jnp.where(upd, dcol, vmin)
            iml = jnp.where(upd, lane + float(col), iml)
    dmin = jnp.min(vmin, axis=1, keepdims=True)
    imin = jnp.min(jnp.where(vmin == dmin, iml, float(N_E)), axis=1,
                   keepdims=True)
    idx_ref[...] = imin.astype(jnp.int32)
    s = jnp.sum(dmin).reshape(1, 1)

    @pl.when(i == 0)
    def _():
        dsum_ref[...] = s

    @pl.when(i != 0)
    def _():
        dsum_ref[...] += s


def _tc_argmin(z_flat, sz, sw, wt):
    return pl.pallas_call(
        _argmin_body,
        grid=(NT,),
        in_specs=[
            pl.BlockSpec((BM, E_DIM), lambda i: (i, 0)),
            pl.BlockSpec((BM, 1), lambda i: (i, 0)),
            pl.BlockSpec((1, N_E), lambda i: (0, 0)),
            pl.BlockSpec((E_DIM, N_E), lambda i: (0, 0)),
        ],
        out_specs=[
            pl.BlockSpec((BM, 1), lambda i: (i, 0)),
            pl.BlockSpec((1, 1), lambda i: (0, 0)),
        ],
        out_shape=[
            jax.ShapeDtypeStruct((N_TOK, 1), jnp.int32),
            jax.ShapeDtypeStruct((1, 1), jnp.float32),
        ],
    )(z_flat, sz, sw, wt)


_B_PER_W = N_TOK // 32
_IDX_ROWS = N_TOK // 128


def _sc_body(w_hbm, idx_hbm, zq_hbm, uniq_hbm,
             idx_v, rows_v, sidx_v, ones_v, zeros_v, slice_v, cnt_v,
             table_sh, sem):
    cid = lax.axis_index("c")
    sid = lax.axis_index("s")
    wid = cid * 16 + sid

    pltpu.sync_copy(idx_hbm.at[pl.ds(wid * 2, 2)], idx_v)
    c0 = pltpu.async_copy(w_hbm.at[idx_v.at[0]], rows_v.at[pl.ds(0, 128)],
                          sem)
    c1 = pltpu.async_copy(w_hbm.at[idx_v.at[1]], rows_v.at[pl.ds(128, 128)],
                          sem)
    c0.wait()
    c1.wait()
    pltpu.sync_copy(rows_v, zq_hbm.at[pl.ds(wid * _B_PER_W, _B_PER_W)])

    @pl.when(cid == 0)
    def _():
        for k in range(512 // 16):
            ones_v[pl.ds(k * 16, 16)] = jnp.ones((16,), jnp.int32)
            zeros_v[pl.ds(k * 16, 16)] = jnp.zeros((16,), jnp.int32)
        pltpu.sync_copy(zeros_v, table_sh.at[pl.ds(sid * 512, 512)])
        plsc.subcore_barrier()
        pltpu.sync_copy(idx_hbm.at[pl.ds(sid * 4, 4)], sidx_v)
        for j in range(4):
            pltpu.sync_copy(ones_v.at[pl.ds(j * 128, 128)],
                            table_sh.at[sidx_v.at[j]], add=True)
        plsc.subcore_barrier()
        pltpu.sync_copy(table_sh.at[pl.ds(sid * 512, 512)], slice_v)
        total = jnp.zeros((16,), jnp.int32)
        for k in range(512 // 16):
            x = slice_v[pl.ds(k * 16, 16)]
            total = total + jnp.minimum(x, jnp.ones((16,), jnp.int32))
        cnt_v[...] = total
        pltpu.sync_copy(cnt_v, uniq_hbm.at[sid])


@functools.cache
def _sc_gather_kernel():
    return pl.kernel(
        _sc_body,
        out_type=[
            jax.ShapeDtypeStruct((N_TOK, E_DIM), jnp.float32),
            jax.ShapeDtypeStruct((16, 16), jnp.int32),
        ],
        mesh=plsc.VectorSubcoreMesh(core_axis_name="c",
                                    subcore_axis_name="s"),
        scratch_types=[
            pltpu.VMEM((2, 128), jnp.int32),
            pltpu.VMEM((_B_PER_W, E_DIM), jnp.float32),
            pltpu.VMEM((4, 128), jnp.int32),
            pltpu.VMEM((512,), jnp.int32),
            pltpu.VMEM((512,), jnp.int32),
            pltpu.VMEM((512,), jnp.int32),
            pltpu.VMEM((16,), jnp.int32),
            pltpu.VMEM_SHARED((N_E,), jnp.int32),
            pltpu.SemaphoreType.DMA,
        ],
    )


def _sc_gather(w, idx2d):
    return _sc_gather_kernel()(w, idx2d)


def kernel(z, W):
    zp = jnp.transpose(z, (0, 2, 3, 4, 1))
    z_flat = zp.reshape(-1, E_DIM)
    sz = jnp.sum(z_flat ** 2, axis=1, keepdims=True)
    sw = jnp.sum(W ** 2, axis=1).reshape(1, N_E)
    z_bf = z_flat.astype(jnp.bfloat16)
    wt_bf = (2.0 * W).astype(jnp.bfloat16).T

    idx2d, dsum = _tc_argmin(z_bf, sz, sw, wt_bf)
    idx = idx2d.reshape(N_TOK)

    zq_flat, uniq_rows = _sc_gather(W, idx.reshape(_IDX_ROWS, 128))
    unique = jnp.sum(uniq_rows)

    m = dsum[0, 0] / jnp.float32(N_TOK * E_DIM)
    loss = jnp.float32(BETA) * m + m

    zq_st = z_flat + (zq_flat - z_flat)
    z_q = jnp.transpose(zq_st.reshape(zp.shape), (0, 4, 1, 2, 3))
    return (z_q, loss, unique, idx)
